# baseline (speedup 1.0000x reference)
import functools

import jax
import jax.numpy as jnp
from jax.experimental import pallas as pl
from jax.experimental.pallas import tpu as pltpu


def _se_gate_body(xa_ref, xb_ref, onesa_ref, onesb_ref, w1t_ref, w2t_ref,
                  g_ref, *, k):
    for i in range(k):
        pooled = jax.lax.dot_general(
            xa_ref[i], onesa_ref[...], (((1,), (0,)), ((), ())),
            preferred_element_type=jnp.float32)
        pooled += jax.lax.dot_general(
            xb_ref[i], onesb_ref[...], (((1,), (0,)), ((), ())),
            preferred_element_type=jnp.float32)
        hidden = jnp.maximum(
            jax.lax.dot_general(w1t_ref[...], pooled,
                                (((1,), (0,)), ((), ())),
                                preferred_element_type=jnp.float32), 0.0)
        g_ref[i] = jax.nn.sigmoid(
            jax.lax.dot_general(w2t_ref[...], hidden,
                                (((1,), (0,)), ((), ())),
                                preferred_element_type=jnp.float32))


def kernel(x_nchw, w1, w2):
    B, C, H, W = x_nchw.shape
    Cr = w1.shape[1]
    HW = H * W
    x_flat = x_nchw.reshape(B, C, HW)

    hw_main = (HW // 128) * 128
    hw_tail = HW - hw_main
    k = 4 if B % 4 == 0 else 1

    in_specs = [
        pl.BlockSpec((k, C, hw_main), lambda b: (b, 0, 0)),
        pl.BlockSpec((k, C, hw_tail), lambda b: (b, 0, 0)),
        pl.BlockSpec((hw_main, 128), lambda b: (0, 0)),
        pl.BlockSpec((hw_tail, 128), lambda b: (0, 0)),
        pl.BlockSpec((Cr, C), lambda b: (0, 0)),
        pl.BlockSpec((C, Cr), lambda b: (0, 0)),
    ]
    inv_hw = 1.0 / float(HW)
    gates = pl.pallas_call(
        functools.partial(_se_gate_body, k=k),
        out_shape=jax.ShapeDtypeStruct((B, C, 128), jnp.float32),
        grid=(B // k,),
        in_specs=in_specs,
        out_specs=pl.BlockSpec((k, C, 128), lambda b: (b, 0, 0)),
        compiler_params=pltpu.CompilerParams(
            dimension_semantics=("arbitrary",),
            vmem_limit_bytes=56 * 1024 * 1024),
    )(x_flat, x_flat[:, :, hw_main:],
      jnp.full((hw_main, 128), inv_hw, jnp.float32),
      jnp.full((hw_tail, 128), inv_hw, jnp.float32),
      w1.T, w2.T)

    return x_nchw * gates[:, :, :1].reshape(B, C, 1, 1)


# speedup vs baseline: 1.0174x; 1.0174x over previous
import jax
import jax.numpy as jnp
from jax.experimental import pallas as pl
from jax.experimental.pallas import tpu as pltpu


def _se_gate_body(x_ref, onesw_ref, w1t_ref, w2t_ref, g_ref, *, k):
    for i in range(k):
        pooled = jax.lax.dot_general(
            x_ref[i], onesw_ref[...], (((1,), (0,)), ((), ())),
            preferred_element_type=jnp.float32)
        hidden = jnp.maximum(
            jax.lax.dot_general(w1t_ref[...], pooled,
                                (((1,), (0,)), ((), ())),
                                preferred_element_type=jnp.float32), 0.0)
        g_ref[i] = jax.nn.sigmoid(
            jax.lax.dot_general(w2t_ref[...], hidden,
                                (((1,), (0,)), ((), ())),
                                preferred_element_type=jnp.float32))
import functools


def kernel(x_nchw, w1, w2):
    B, C, H, W = x_nchw.shape
    Cr = w1.shape[1]
    HW = H * W
    x_flat = x_nchw.reshape(B, C, HW)
    k = 8 if B % 8 == 0 else 1

    gates = pl.pallas_call(
        functools.partial(_se_gate_body, k=k),
        out_shape=jax.ShapeDtypeStruct((B, C, 128), jnp.float32),
        grid=(B // k,),
        in_specs=[
            pl.BlockSpec((k, C, HW), lambda b: (b, 0, 0)),
            pl.BlockSpec((HW, 128), lambda b: (0, 0)),
            pl.BlockSpec((Cr, C), lambda b: (0, 0)),
            pl.BlockSpec((C, Cr), lambda b: (0, 0)),
        ],
        out_specs=pl.BlockSpec((k, C, 128), lambda b: (b, 0, 0)),
        compiler_params=pltpu.CompilerParams(
            dimension_semantics=("arbitrary",),
            vmem_limit_bytes=56 * 1024 * 1024),
    )(x_flat, jnp.full((HW, 128), 1.0 / float(HW), jnp.float32), w1.T, w2.T)

    return x_nchw * gates[:, :, :1].reshape(B, C, 1, 1)
